# baseline (speedup 1.0000x reference)
import functools

import jax
import jax.numpy as jnp
from jax import lax
from jax.experimental import pallas as pl
from jax.experimental.pallas import tpu as pltpu
from jax.experimental.pallas import tpu_sc as plsc

N = 10000
E = 320000
D = 128
G = 64

NC = 2
NS = 16
NW = NC * NS
CHUNK = 96
NCF = 121
NCS = 88
NBUF = 3
NPIDX = (NS * NCF + (NS - 1) * NCS + NCF) * CHUNK
NPAD = 8
RPS = 624
TAIL = N - NS * RPS
ZR = 208
ZPASS = RPS // ZR

_mesh = plsc.VectorSubcoreMesh(core_axis_name="c", subcore_axis_name="s")


@functools.partial(
    pl.kernel,
    out_type=jax.ShapeDtypeStruct((NC * N, D), jnp.float32),
    mesh=_mesh,
    scratch_types=[
        pltpu.VMEM((NCF * CHUNK,), jnp.int32),
        pltpu.VMEM((NBUF, CHUNK), jnp.int32),
        pltpu.VMEM((NBUF, CHUNK), jnp.int32),
        pltpu.VMEM((NBUF, CHUNK, D), jnp.float32),
        pltpu.VMEM_SHARED((N + NPAD, D), jnp.float32),
        pltpu.SemaphoreType.DMA,
        pltpu.SemaphoreType.DMA((NBUF,)),
        pltpu.SemaphoreType.DMA((NBUF,)),
    ],
)
def _sc_segment_sum(x_hbm, pidx_hbm, z_hbm, out_hbm, pidx, sidx, didx, rows, acc, isem, gsem, ssem):
    c = lax.axis_index("c")
    s = lax.axis_index("s")

    nch = jnp.where(c == 0, NCF, NCS)
    base = jnp.where(c == 0, s * NCF, NS * NCF + s * NCS)
    cp_i = pltpu.async_copy(pidx_hbm.at[pl.ds(base * CHUNK, NCF * CHUNK)], pidx, isem)

    for k in range(ZPASS):
        pltpu.sync_copy(z_hbm, acc.at[pl.ds(s * RPS + k * ZR, ZR)])

    @pl.when(s == NS - 1)
    def _zero_tail():
        pltpu.sync_copy(z_hbm.at[pl.ds(0, TAIL)], acc.at[pl.ds(NS * RPS, TAIL)])

    cp_i.wait()

    def _unpack(kn, b):
        for i in range(CHUNK // 16):
            v = pidx[pl.ds(kn * CHUNK + i * 16, 16)]
            sidx[b, pl.ds(i * 16, 16)] = v & 0xFFFF
            didx[b, pl.ds(i * 16, 16)] = v >> 16

    def _gather(b):
        pltpu.async_copy(x_hbm.at[sidx.at[b]], rows.at[b], gsem.at[b])

    def _wait(sem, b):
        pltpu.make_async_copy(x_hbm.at[pl.ds(0, CHUNK)], rows.at[b], sem.at[b]).wait()

    def _scatter(b):
        pltpu.async_copy(rows.at[b], acc.at[didx.at[b]], ssem.at[b], add=True)

    for b in range(2):
        _unpack(b, b)
        _gather(b)
    _wait(gsem, 0)
    plsc.subcore_barrier()
    _scatter(0)
    _unpack(2, 2)
    _gather(2)

    def _round(j, carry):
        for p in range(NBUF):
            k = j * NBUF + 1 + p
            b = (1 + p) % NBUF
            bp = p
            _wait(gsem, b)
            _scatter(b)
            _wait(ssem, bp)

            @pl.when(k + 2 < nch)
            def _prefetch():
                _unpack(k + 2, bp)
                _gather(bp)

        return carry

    lax.fori_loop(0, (nch - 1) // NBUF, _round, 0)

    _wait(ssem, 0)
    plsc.subcore_barrier()

    pltpu.sync_copy(
        acc.at[pl.ds(s * RPS, RPS)],
        out_hbm.at[pl.ds(c * N + s * RPS, RPS)],
    )

    @pl.when(s == NS - 1)
    def _write_tail():
        pltpu.sync_copy(
            acc.at[pl.ds(NS * RPS, TAIL)],
            out_hbm.at[pl.ds(c * N + NS * RPS, TAIL)],
        )


def _layer_body(pp_ref, x_ref, wrel_ref, wroot_ref, b_ref, o_ref):
    agg = pp_ref[0:N, :] + pp_ref[N : 2 * N, :]
    h = (
        jnp.dot(agg, wrel_ref[...], preferred_element_type=jnp.float32)
        + jnp.dot(x_ref[...], wroot_ref[...], preferred_element_type=jnp.float32)
        + b_ref[...]
    )
    o_ref[...] = jnp.maximum(h, 0.0)


_layer = pl.pallas_call(
    _layer_body,
    out_shape=jax.ShapeDtypeStruct((N, D), jnp.float32),
)


def _final_body(pp_ref, x_ref, wrel_ref, wroot_ref, b_ref, batch_ref, wlin_ref, blin_ref, o_ref):
    agg = pp_ref[0:N, :] + pp_ref[N : 2 * N, :]
    h = (
        jnp.dot(agg, wrel_ref[...], preferred_element_type=jnp.float32)
        + jnp.dot(x_ref[...], wroot_ref[...], preferred_element_type=jnp.float32)
        + b_ref[...]
    )
    bt = batch_ref[...]
    gids = lax.broadcasted_iota(jnp.int32, (G, N), 0)
    onehot_t = (gids == bt).astype(jnp.float32)
    sums = jnp.dot(onehot_t, h, preferred_element_type=jnp.float32)
    counts = jnp.sum(onehot_t, axis=1, keepdims=True)
    pooled = sums / jnp.maximum(counts, 1.0)
    o_ref[...] = (
        jnp.dot(pooled, wlin_ref[...], preferred_element_type=jnp.float32)
        + blin_ref[...]
    )


_final = pl.pallas_call(
    _final_body,
    out_shape=jax.ShapeDtypeStruct((G, D), jnp.float32),
)


def kernel(x, edge_index, batch, dropout_prob, W_rel1, W_root1, W_rel2, W_root2, W_rel3, W_root3, W_lin, b1, b2, b3, b_lin):
    src = edge_index[0]
    dst = edge_index[1]
    packed = src | (dst << 16)
    pad = jnp.full((NPIDX - E,), N << 16, jnp.int32)
    pidx_flat = jnp.concatenate([packed, pad])
    zeros = jnp.zeros((ZR, D), jnp.float32)
    batch2 = batch.reshape(1, N)

    p1 = _sc_segment_sum(x, pidx_flat, zeros)
    h1 = _layer(p1, x, W_rel1, W_root1, b1.reshape(1, D))
    p2 = _sc_segment_sum(h1, pidx_flat, zeros)
    h2 = _layer(p2, h1, W_rel2, W_root2, b2.reshape(1, D))
    p3 = _sc_segment_sum(h2, pidx_flat, zeros)
    out = _final(p3, h2, W_rel3, W_root3, b3.reshape(1, D), batch2, W_lin, b_lin.reshape(1, D))
    return out


# speedup vs baseline: 1.0156x; 1.0156x over previous
import functools

import jax
import jax.numpy as jnp
from jax import lax
from jax.experimental import pallas as pl
from jax.experimental.pallas import tpu as pltpu
from jax.experimental.pallas import tpu_sc as plsc

N = 10000
E = 320000
D = 128
G = 64

NC = 2
NS = 16
NW = NC * NS
CHUNK = 96
NCF = 124
NCS = 85
NBUF = 3
NPIDX = (NS * NCF + (NS - 1) * NCS + NCF) * CHUNK
NPAD = 8
RPS = 624
TAIL = N - NS * RPS
ZR = 208
ZPASS = RPS // ZR

_mesh = plsc.VectorSubcoreMesh(core_axis_name="c", subcore_axis_name="s")


@functools.partial(
    pl.kernel,
    out_type=jax.ShapeDtypeStruct((NC * N, D), jnp.float32),
    mesh=_mesh,
    scratch_types=[
        pltpu.VMEM((NCF * CHUNK,), jnp.int32),
        pltpu.VMEM((NBUF, CHUNK), jnp.int32),
        pltpu.VMEM((NBUF, CHUNK), jnp.int32),
        pltpu.VMEM((NBUF, CHUNK, D), jnp.float32),
        pltpu.VMEM_SHARED((N + NPAD, D), jnp.float32),
        pltpu.SemaphoreType.DMA,
        pltpu.SemaphoreType.DMA((NBUF,)),
        pltpu.SemaphoreType.DMA((NBUF,)),
    ],
)
def _sc_segment_sum(x_hbm, pidx_hbm, z_hbm, out_hbm, pidx, sidx, didx, rows, acc, isem, gsem, ssem):
    c = lax.axis_index("c")
    s = lax.axis_index("s")

    nch = jnp.where(c == 0, NCF, NCS)
    base = jnp.where(c == 0, s * NCF, NS * NCF + s * NCS)
    cp_i = pltpu.async_copy(pidx_hbm.at[pl.ds(base * CHUNK, NCF * CHUNK)], pidx, isem)

    for k in range(ZPASS):
        pltpu.sync_copy(z_hbm, acc.at[pl.ds(s * RPS + k * ZR, ZR)])

    @pl.when(s == NS - 1)
    def _zero_tail():
        pltpu.sync_copy(z_hbm.at[pl.ds(0, TAIL)], acc.at[pl.ds(NS * RPS, TAIL)])

    cp_i.wait()

    def _unpack(kn, b):
        for i in range(CHUNK // 16):
            v = pidx[pl.ds(kn * CHUNK + i * 16, 16)]
            sidx[b, pl.ds(i * 16, 16)] = v & 0xFFFF
            didx[b, pl.ds(i * 16, 16)] = v >> 16

    def _gather(b):
        pltpu.async_copy(x_hbm.at[sidx.at[b]], rows.at[b], gsem.at[b])

    def _wait(sem, b):
        pltpu.make_async_copy(x_hbm.at[pl.ds(0, CHUNK)], rows.at[b], sem.at[b]).wait()

    def _scatter(b):
        pltpu.async_copy(rows.at[b], acc.at[didx.at[b]], ssem.at[b], add=True)

    for b in range(2):
        _unpack(b, b)
        _gather(b)
    _wait(gsem, 0)
    plsc.subcore_barrier()
    _scatter(0)
    _unpack(2, 2)
    _gather(2)

    def _round(j, carry):
        for p in range(NBUF):
            k = j * NBUF + 1 + p
            b = (1 + p) % NBUF
            bp = p
            _wait(gsem, b)
            _scatter(b)
            _wait(ssem, bp)

            @pl.when(k + 2 < nch)
            def _prefetch():
                _unpack(k + 2, bp)
                _gather(bp)

        return carry

    lax.fori_loop(0, (nch - 1) // NBUF, _round, 0)

    _wait(ssem, 0)
    plsc.subcore_barrier()

    pltpu.sync_copy(
        acc.at[pl.ds(s * RPS, RPS)],
        out_hbm.at[pl.ds(c * N + s * RPS, RPS)],
    )

    @pl.when(s == NS - 1)
    def _write_tail():
        pltpu.sync_copy(
            acc.at[pl.ds(NS * RPS, TAIL)],
            out_hbm.at[pl.ds(c * N + NS * RPS, TAIL)],
        )


def _layer_body(pp_ref, x_ref, wrel_ref, wroot_ref, b_ref, o_ref):
    agg = pp_ref[0:N, :] + pp_ref[N : 2 * N, :]
    h = (
        jnp.dot(agg, wrel_ref[...], preferred_element_type=jnp.float32)
        + jnp.dot(x_ref[...], wroot_ref[...], preferred_element_type=jnp.float32)
        + b_ref[...]
    )
    o_ref[...] = jnp.maximum(h, 0.0)


_layer = pl.pallas_call(
    _layer_body,
    out_shape=jax.ShapeDtypeStruct((N, D), jnp.float32),
)


def _final_body(pp_ref, x_ref, wrel_ref, wroot_ref, b_ref, batch_ref, wlin_ref, blin_ref, o_ref):
    agg = pp_ref[0:N, :] + pp_ref[N : 2 * N, :]
    h = (
        jnp.dot(agg, wrel_ref[...], preferred_element_type=jnp.float32)
        + jnp.dot(x_ref[...], wroot_ref[...], preferred_element_type=jnp.float32)
        + b_ref[...]
    )
    bt = batch_ref[...]
    gids = lax.broadcasted_iota(jnp.int32, (G, N), 0)
    onehot_t = (gids == bt).astype(jnp.float32)
    sums = jnp.dot(onehot_t, h, preferred_element_type=jnp.float32)
    counts = jnp.sum(onehot_t, axis=1, keepdims=True)
    pooled = sums / jnp.maximum(counts, 1.0)
    o_ref[...] = (
        jnp.dot(pooled, wlin_ref[...], preferred_element_type=jnp.float32)
        + blin_ref[...]
    )


_final = pl.pallas_call(
    _final_body,
    out_shape=jax.ShapeDtypeStruct((G, D), jnp.float32),
)


def kernel(x, edge_index, batch, dropout_prob, W_rel1, W_root1, W_rel2, W_root2, W_rel3, W_root3, W_lin, b1, b2, b3, b_lin):
    src = edge_index[0]
    dst = edge_index[1]
    packed = src | (dst << 16)
    pad = jnp.full((NPIDX - E,), N << 16, jnp.int32)
    pidx_flat = jnp.concatenate([packed, pad])
    zeros = jnp.zeros((ZR, D), jnp.float32)
    batch2 = batch.reshape(1, N)

    p1 = _sc_segment_sum(x, pidx_flat, zeros)
    h1 = _layer(p1, x, W_rel1, W_root1, b1.reshape(1, D))
    p2 = _sc_segment_sum(h1, pidx_flat, zeros)
    h2 = _layer(p2, h1, W_rel2, W_root2, b2.reshape(1, D))
    p3 = _sc_segment_sum(h2, pidx_flat, zeros)
    out = _final(p3, h2, W_rel3, W_root3, b3.reshape(1, D), batch2, W_lin, b_lin.reshape(1, D))
    return out


# speedup vs baseline: 1.0989x; 1.0820x over previous
import functools

import jax
import jax.numpy as jnp
from jax import lax
from jax.experimental import pallas as pl
from jax.experimental.pallas import tpu as pltpu
from jax.experimental.pallas import tpu_sc as plsc

N = 10000
E = 320000
D = 128
G = 64

NC = 2
NS = 16
NW = NC * NS
CHUNK = 112
NCF = 106
NCS = 73
NBUF = 3
NPIDX = (NS * NCF + NS * NCS) * CHUNK
NPAD = 8
RPS = 624
TAIL = N - NS * RPS
ZR = 208
ZPASS = RPS // ZR

_mesh = plsc.VectorSubcoreMesh(core_axis_name="c", subcore_axis_name="s")


@functools.partial(
    pl.kernel,
    out_type=jax.ShapeDtypeStruct((NC * N, D), jnp.float32),
    mesh=_mesh,
    scratch_types=[
        pltpu.VMEM((NBUF, CHUNK), jnp.int32),
        pltpu.VMEM((NBUF, CHUNK), jnp.int32),
        pltpu.VMEM((NBUF, CHUNK), jnp.int32),
        pltpu.VMEM((NBUF, CHUNK, D), jnp.float32),
        pltpu.VMEM_SHARED((N + NPAD, D), jnp.float32),
        pltpu.SemaphoreType.DMA((NBUF,)),
        pltpu.SemaphoreType.DMA((NBUF,)),
        pltpu.SemaphoreType.DMA((NBUF,)),
    ],
)
def _sc_segment_sum(x_hbm, pidx_hbm, z_hbm, out_hbm, pbuf, sidx, didx, rows, acc, psem, gsem, ssem):
    c = lax.axis_index("c")
    s = lax.axis_index("s")

    nch = jnp.where(c == 0, NCF, NCS)
    base = jnp.where(c == 0, s * NCF, NS * NCF + s * NCS)

    def _fetch(kn, b):
        pltpu.async_copy(
            pidx_hbm.at[pl.ds((base + kn) * CHUNK, CHUNK)], pbuf.at[b], psem.at[b]
        )

    def _waitp(b):
        pltpu.make_async_copy(
            pidx_hbm.at[pl.ds(0, CHUNK)], pbuf.at[b], psem.at[b]
        ).wait()

    for b in range(NBUF):
        _fetch(b, b)

    for k in range(ZPASS):
        pltpu.sync_copy(z_hbm, acc.at[pl.ds(s * RPS + k * ZR, ZR)])

    @pl.when(s == NS - 1)
    def _zero_tail():
        pltpu.sync_copy(z_hbm.at[pl.ds(0, TAIL)], acc.at[pl.ds(NS * RPS, TAIL)])

    def _unpack(b):
        for i in range(CHUNK // 16):
            v = pbuf[b, pl.ds(i * 16, 16)]
            sidx[b, pl.ds(i * 16, 16)] = v & 0xFFFF
            didx[b, pl.ds(i * 16, 16)] = v >> 16

    def _gather(b):
        pltpu.async_copy(x_hbm.at[sidx.at[b]], rows.at[b], gsem.at[b])

    def _wait(sem, b):
        pltpu.make_async_copy(x_hbm.at[pl.ds(0, CHUNK)], rows.at[b], sem.at[b]).wait()

    def _scatter(b):
        pltpu.async_copy(rows.at[b], acc.at[didx.at[b]], ssem.at[b], add=True)

    for b in range(2):
        _waitp(b)
        _unpack(b)
        _gather(b)
    _wait(gsem, 0)
    plsc.subcore_barrier()
    _scatter(0)
    _waitp(2)
    _unpack(2)
    _gather(2)
    _fetch(3, 0)

    def _round(j, carry):
        for p in range(NBUF):
            k = j * NBUF + 1 + p
            b = (1 + p) % NBUF
            bp = p
            _wait(gsem, b)
            _scatter(b)
            _wait(ssem, bp)

            @pl.when(k + 2 < nch)
            def _prefetch():
                _waitp(bp)
                _unpack(bp)
                _gather(bp)

            @pl.when(k + 3 < nch)
            def _fetch_next():
                _fetch(k + 3, b)

        return carry

    lax.fori_loop(0, (nch - 1) // NBUF, _round, 0)

    _wait(ssem, 0)
    plsc.subcore_barrier()

    pltpu.sync_copy(
        acc.at[pl.ds(s * RPS, RPS)],
        out_hbm.at[pl.ds(c * N + s * RPS, RPS)],
    )

    @pl.when(s == NS - 1)
    def _write_tail():
        pltpu.sync_copy(
            acc.at[pl.ds(NS * RPS, TAIL)],
            out_hbm.at[pl.ds(c * N + NS * RPS, TAIL)],
        )


def _layer_body(pp_ref, x_ref, wrel_ref, wroot_ref, b_ref, o_ref):
    agg = pp_ref[0:N, :] + pp_ref[N : 2 * N, :]
    h = (
        jnp.dot(agg, wrel_ref[...], preferred_element_type=jnp.float32)
        + jnp.dot(x_ref[...], wroot_ref[...], preferred_element_type=jnp.float32)
        + b_ref[...]
    )
    o_ref[...] = jnp.maximum(h, 0.0)


_layer = pl.pallas_call(
    _layer_body,
    out_shape=jax.ShapeDtypeStruct((N, D), jnp.float32),
)


def _final_body(pp_ref, x_ref, wrel_ref, wroot_ref, b_ref, batch_ref, wlin_ref, blin_ref, o_ref):
    agg = pp_ref[0:N, :] + pp_ref[N : 2 * N, :]
    h = (
        jnp.dot(agg, wrel_ref[...], preferred_element_type=jnp.float32)
        + jnp.dot(x_ref[...], wroot_ref[...], preferred_element_type=jnp.float32)
        + b_ref[...]
    )
    bt = batch_ref[...]
    gids = lax.broadcasted_iota(jnp.int32, (G, N), 0)
    onehot_t = (gids == bt).astype(jnp.float32)
    sums = jnp.dot(onehot_t, h, preferred_element_type=jnp.float32)
    counts = jnp.sum(onehot_t, axis=1, keepdims=True)
    pooled = sums / jnp.maximum(counts, 1.0)
    o_ref[...] = (
        jnp.dot(pooled, wlin_ref[...], preferred_element_type=jnp.float32)
        + blin_ref[...]
    )


_final = pl.pallas_call(
    _final_body,
    out_shape=jax.ShapeDtypeStruct((G, D), jnp.float32),
)


def kernel(x, edge_index, batch, dropout_prob, W_rel1, W_root1, W_rel2, W_root2, W_rel3, W_root3, W_lin, b1, b2, b3, b_lin):
    src = edge_index[0]
    dst = edge_index[1]
    packed = src | (dst << 16)
    pad = jnp.full((NPIDX - E,), N << 16, jnp.int32)
    pidx_flat = jnp.concatenate([packed, pad])
    zeros = jnp.zeros((ZR, D), jnp.float32)
    batch2 = batch.reshape(1, N)

    p1 = _sc_segment_sum(x, pidx_flat, zeros)
    h1 = _layer(p1, x, W_rel1, W_root1, b1.reshape(1, D))
    p2 = _sc_segment_sum(h1, pidx_flat, zeros)
    h2 = _layer(p2, h1, W_rel2, W_root2, b2.reshape(1, D))
    p3 = _sc_segment_sum(h2, pidx_flat, zeros)
    out = _final(p3, h2, W_rel3, W_root3, b3.reshape(1, D), batch2, W_lin, b_lin.reshape(1, D))
    return out
